# baseline (speedup 1.0000x reference)
import functools

import jax
import jax.numpy as jnp
from jax import lax
from jax.experimental import pallas as pl
from jax.experimental.pallas import tpu as pltpu
from jax.experimental.pallas import tpu_sc as plsc

_NBUF = 3
_NCHUNK = 2


def _sc_gather(feats, ids3, b_base, bpc):
    B, C, H, W = feats.shape
    KH = ids3.shape[1]
    K = KH * 128
    info = plsc.get_sparse_core_info()
    NC, NS, L = info.num_cores, info.num_subcores, info.num_lanes
    NW = NC * NS
    assert NW % bpc == 0
    WPB = NW // bpc
    CPW = C // WPB
    assert CPW * WPB == C and CPW % _NBUF == 0
    mesh = plsc.VectorSubcoreMesh(core_axis_name="c", subcore_axis_name="s")

    @functools.partial(
        pl.kernel,
        mesh=mesh,
        out_type=jax.ShapeDtypeStruct((bpc, C, K), jnp.float32),
        scratch_types=[
            pltpu.VMEM((KH, 128), jnp.int32),
            [pltpu.VMEM((H, W), jnp.float32) for _ in range(_NBUF)],
            [pltpu.VMEM((K,), jnp.float32) for _ in range(_NBUF)],
            [pltpu.SemaphoreType.DMA for _ in range(_NBUF)],
            [pltpu.SemaphoreType.DMA for _ in range(_NBUF)],
        ],
        compiler_params=pltpu.CompilerParams(
            needs_layout_passes=False, use_tc_tiling_on_sc=True),
    )
    def gather_kernel(feats_hbm, ids_hbm, out_hbm, ids_v, rows, outs,
                      sems_in, sems_out):
        wid = lax.axis_index("s") * NC + lax.axis_index("c")
        bo = wid // WPB
        b = b_base + bo
        c0 = (wid % WPB) * CPW
        pltpu.sync_copy(ids_hbm.at[b], ids_v)
        for p in range(_NBUF - 1):
            pltpu.async_copy(feats_hbm.at[b, c0 + p], rows[p], sems_in[p])

        def gather_row(p, c):
            pltpu.make_async_copy(feats_hbm.at[b, c], rows[p],
                                  sems_in[p]).wait()

            @pl.when(c + _NBUF - 1 < c0 + CPW)
            def _():
                pltpu.async_copy(feats_hbm.at[b, c + _NBUF - 1],
                                 rows[(p + _NBUF - 1) % _NBUF],
                                 sems_in[(p + _NBUF - 1) % _NBUF])

            ob = outs[p]

            @pl.when(c - _NBUF >= c0)
            def _():
                pltpu.make_async_copy(ob, out_hbm.at[bo, c],
                                      sems_out[p]).wait()

            row = rows[p]
            G = 8
            for j0 in range(0, K // L, G):
                idxs = [ids_v[(j0 + t) // 8, pl.ds(((j0 + t) % 8) * L, L)]
                        for t in range(G)]
                vals = [plsc.load_gather(
                            row, [jnp.right_shift(ix, 7),
                                  jnp.bitwise_and(ix, 127)])
                        for ix in idxs]
                for t in range(G):
                    ob[pl.ds((j0 + t) * L, L)] = vals[t]
            pltpu.async_copy(ob, out_hbm.at[bo, c], sems_out[p])

        def chan_body(ci, _):
            c = c0 + ci * _NBUF
            for p in range(_NBUF):
                gather_row(p, c + p)
            return 0

        lax.fori_loop(0, CPW // _NBUF, chan_body, 0)
        for p in range(_NBUF):
            pltpu.make_async_copy(outs[p], out_hbm.at[bo, c0],
                                  sems_out[p]).wait()

    return gather_kernel(feats, ids3)


def _mlp_body(g_ref, w1_ref, b1_ref, w2_ref, b2_ref, out_ref):
    g = g_ref[0]
    h = lax.dot_general(g, w1_ref[...], (((0,), (0,)), ((), ())),
                        preferred_element_type=jnp.float32)
    h = jnp.maximum(h + b1_ref[...], 0.0)
    p = jnp.dot(h, w2_ref[...], preferred_element_type=jnp.float32)
    p = p + b2_ref[...]
    nrm = jnp.sqrt(jnp.sum(p * p, axis=1, keepdims=True))
    out_ref[...] = p / jnp.maximum(nrm, 1e-12)


def _mlp_alias_body(g_ref, w1_ref, b1_ref, w2_ref, b2_ref, prev_ref, out_ref):
    del prev_ref
    _mlp_body(g_ref, w1_ref, b1_ref, w2_ref, b2_ref, out_ref)


def _tc_mlp(gT, W1, b1, W2, b2, nrows, block_base, prev=None):
    bpc, C, K = gT.shape
    P = W1.shape[1]
    b1r = b1.reshape(1, P)
    b2r = b2.reshape(1, P)
    in_specs = [
        pl.BlockSpec((1, C, K), lambda b: (b, 0, 0)),
        pl.BlockSpec((C, P), lambda b: (0, 0)),
        pl.BlockSpec((1, P), lambda b: (0, 0)),
        pl.BlockSpec((P, P), lambda b: (0, 0)),
        pl.BlockSpec((1, P), lambda b: (0, 0)),
    ]
    args = [gT, W1, b1r, W2, b2r]
    kwargs = {}
    body = _mlp_body
    if prev is not None:
        in_specs.append(pl.BlockSpec(memory_space=pl.ANY))
        args.append(prev)
        kwargs["input_output_aliases"] = {5: 0}
        body = _mlp_alias_body

    return pl.pallas_call(
        body,
        grid=(bpc,),
        in_specs=in_specs,
        out_specs=pl.BlockSpec((K, P), lambda b: (block_base + b, 0)),
        out_shape=jax.ShapeDtypeStruct((nrows, P), jnp.float32),
        **kwargs,
    )(*args)


def kernel(feats, patch_ids, num_patches, W1, b1, W2, b2):
    B, C, H, W = feats.shape
    K = patch_ids.shape[1]
    ids3 = patch_ids.reshape(B, K // 128, 128)
    bpc = B // _NCHUNK
    gs = [_sc_gather(feats, ids3, i * bpc, bpc) for i in range(_NCHUNK)]
    p = _tc_mlp(gs[0], W1, b1, W2, b2, B * K, 0)
    for i in range(1, _NCHUNK):
        p = _tc_mlp(gs[i], W1, b1, W2, b2, B * K, i * bpc, prev=p)
    return (p, patch_ids)


# speedup vs baseline: 1.0928x; 1.0928x over previous
import functools

import jax
import jax.numpy as jnp
from jax import lax
from jax.experimental import pallas as pl
from jax.experimental.pallas import tpu as pltpu
from jax.experimental.pallas import tpu_sc as plsc

_NBUF = 4
_NCHUNK = 1


def _sc_gather(feats, ids3, b_base, bpc):
    B, C, H, W = feats.shape
    KH = ids3.shape[1]
    K = KH * 128
    info = plsc.get_sparse_core_info()
    NC, NS, L = info.num_cores, info.num_subcores, info.num_lanes
    NW = NC * NS
    assert NW % bpc == 0
    WPB = NW // bpc
    CPW = C // WPB
    assert CPW * WPB == C and CPW % _NBUF == 0
    mesh = plsc.VectorSubcoreMesh(core_axis_name="c", subcore_axis_name="s")

    @functools.partial(
        pl.kernel,
        mesh=mesh,
        out_type=jax.ShapeDtypeStruct((bpc, C, K), jnp.float32),
        scratch_types=[
            pltpu.VMEM((KH, 128), jnp.int32),
            [pltpu.VMEM((H, W), jnp.float32) for _ in range(_NBUF)],
            [pltpu.VMEM((K,), jnp.float32) for _ in range(_NBUF)],
            [pltpu.SemaphoreType.DMA for _ in range(_NBUF)],
            [pltpu.SemaphoreType.DMA for _ in range(_NBUF)],
        ],
        compiler_params=pltpu.CompilerParams(
            needs_layout_passes=False, use_tc_tiling_on_sc=True),
    )
    def gather_kernel(feats_hbm, ids_hbm, out_hbm, ids_v, rows, outs,
                      sems_in, sems_out):
        wid = lax.axis_index("s") * NC + lax.axis_index("c")
        bo = wid // WPB
        b = b_base + bo
        c0 = (wid % WPB) * CPW
        pltpu.sync_copy(ids_hbm.at[b], ids_v)
        for p in range(_NBUF - 1):
            pltpu.async_copy(feats_hbm.at[b, c0 + p], rows[p], sems_in[p])

        def gather_row(p, c):
            pltpu.make_async_copy(feats_hbm.at[b, c], rows[p],
                                  sems_in[p]).wait()

            @pl.when(c + _NBUF - 1 < c0 + CPW)
            def _():
                pltpu.async_copy(feats_hbm.at[b, c + _NBUF - 1],
                                 rows[(p + _NBUF - 1) % _NBUF],
                                 sems_in[(p + _NBUF - 1) % _NBUF])

            ob = outs[p]

            @pl.when(c - _NBUF >= c0)
            def _():
                pltpu.make_async_copy(ob, out_hbm.at[bo, c],
                                      sems_out[p]).wait()

            row = rows[p]
            G = 8
            for j0 in range(0, K // L, G):
                idxs = [ids_v[(j0 + t) // 8, pl.ds(((j0 + t) % 8) * L, L)]
                        for t in range(G)]
                vals = [plsc.load_gather(
                            row, [jnp.right_shift(ix, 7),
                                  jnp.bitwise_and(ix, 127)])
                        for ix in idxs]
                for t in range(G):
                    ob[pl.ds((j0 + t) * L, L)] = vals[t]
            pltpu.async_copy(ob, out_hbm.at[bo, c], sems_out[p])

        def chan_body(ci, _):
            c = c0 + ci * _NBUF
            for p in range(_NBUF):
                gather_row(p, c + p)
            return 0

        lax.fori_loop(0, CPW // _NBUF, chan_body, 0)
        for p in range(_NBUF):
            pltpu.make_async_copy(outs[p], out_hbm.at[bo, c0],
                                  sems_out[p]).wait()

    return gather_kernel(feats, ids3)


def _mlp_body(g_ref, w1_ref, b1_ref, w2_ref, b2_ref, out_ref):
    g = g_ref[0]
    h = lax.dot_general(g, w1_ref[...], (((0,), (0,)), ((), ())),
                        preferred_element_type=jnp.float32)
    h = jnp.maximum(h + b1_ref[...], 0.0)
    p = jnp.dot(h, w2_ref[...], preferred_element_type=jnp.float32)
    p = p + b2_ref[...]
    nrm = jnp.sqrt(jnp.sum(p * p, axis=1, keepdims=True))
    out_ref[...] = p / jnp.maximum(nrm, 1e-12)


def _mlp_alias_body(g_ref, w1_ref, b1_ref, w2_ref, b2_ref, prev_ref, out_ref):
    del prev_ref
    _mlp_body(g_ref, w1_ref, b1_ref, w2_ref, b2_ref, out_ref)


def _tc_mlp(gT, W1, b1, W2, b2, nrows, block_base, prev=None):
    bpc, C, K = gT.shape
    P = W1.shape[1]
    b1r = b1.reshape(1, P)
    b2r = b2.reshape(1, P)
    in_specs = [
        pl.BlockSpec((1, C, K), lambda b: (b, 0, 0)),
        pl.BlockSpec((C, P), lambda b: (0, 0)),
        pl.BlockSpec((1, P), lambda b: (0, 0)),
        pl.BlockSpec((P, P), lambda b: (0, 0)),
        pl.BlockSpec((1, P), lambda b: (0, 0)),
    ]
    args = [gT, W1, b1r, W2, b2r]
    kwargs = {}
    body = _mlp_body
    if prev is not None:
        in_specs.append(pl.BlockSpec(memory_space=pl.ANY))
        args.append(prev)
        kwargs["input_output_aliases"] = {5: 0}
        body = _mlp_alias_body

    return pl.pallas_call(
        body,
        grid=(bpc,),
        in_specs=in_specs,
        out_specs=pl.BlockSpec((K, P), lambda b: (block_base + b, 0)),
        out_shape=jax.ShapeDtypeStruct((nrows, P), jnp.float32),
        **kwargs,
    )(*args)


def kernel(feats, patch_ids, num_patches, W1, b1, W2, b2):
    B, C, H, W = feats.shape
    K = patch_ids.shape[1]
    ids3 = patch_ids.reshape(B, K // 128, 128)
    bpc = B // _NCHUNK
    gs = [_sc_gather(feats, ids3, i * bpc, bpc) for i in range(_NCHUNK)]
    p = _tc_mlp(gs[0], W1, b1, W2, b2, B * K, 0)
    for i in range(1, _NCHUNK):
        p = _tc_mlp(gs[i], W1, b1, W2, b2, B * K, i * bpc, prev=p)
    return (p, patch_ids)


# speedup vs baseline: 1.1008x; 1.0074x over previous
import functools

import jax
import jax.numpy as jnp
from jax import lax
from jax.experimental import pallas as pl
from jax.experimental.pallas import tpu as pltpu
from jax.experimental.pallas import tpu_sc as plsc

_NBUF = 6
_NCHUNK = 1


def _sc_gather(feats, ids3, b_base, bpc):
    B, C, H, W = feats.shape
    KH = ids3.shape[1]
    K = KH * 128
    info = plsc.get_sparse_core_info()
    NC, NS, L = info.num_cores, info.num_subcores, info.num_lanes
    NW = NC * NS
    assert NW % bpc == 0
    WPB = NW // bpc
    CPW = C // WPB
    assert CPW * WPB == C and CPW % _NBUF == 0
    mesh = plsc.VectorSubcoreMesh(core_axis_name="c", subcore_axis_name="s")

    @functools.partial(
        pl.kernel,
        mesh=mesh,
        out_type=jax.ShapeDtypeStruct((bpc, C, K), jnp.float32),
        scratch_types=[
            pltpu.VMEM((KH, 128), jnp.int32),
            [pltpu.VMEM((H, W), jnp.float32) for _ in range(_NBUF)],
            [pltpu.VMEM((K,), jnp.float32) for _ in range(_NBUF)],
            [pltpu.SemaphoreType.DMA for _ in range(_NBUF)],
            [pltpu.SemaphoreType.DMA for _ in range(_NBUF)],
        ],
        compiler_params=pltpu.CompilerParams(
            needs_layout_passes=False, use_tc_tiling_on_sc=True),
    )
    def gather_kernel(feats_hbm, ids_hbm, out_hbm, ids_v, rows, outs,
                      sems_in, sems_out):
        wid = lax.axis_index("s") * NC + lax.axis_index("c")
        bo = wid // WPB
        b = b_base + bo
        c0 = (wid % WPB) * CPW
        pltpu.sync_copy(ids_hbm.at[b], ids_v)
        for p in range(_NBUF - 1):
            pltpu.async_copy(feats_hbm.at[b, c0 + p], rows[p], sems_in[p])

        def gather_row(p, c):
            pltpu.make_async_copy(feats_hbm.at[b, c], rows[p],
                                  sems_in[p]).wait()

            @pl.when(c + _NBUF - 1 < c0 + CPW)
            def _():
                pltpu.async_copy(feats_hbm.at[b, c + _NBUF - 1],
                                 rows[(p + _NBUF - 1) % _NBUF],
                                 sems_in[(p + _NBUF - 1) % _NBUF])

            ob = outs[p]

            @pl.when(c - _NBUF >= c0)
            def _():
                pltpu.make_async_copy(ob, out_hbm.at[bo, c],
                                      sems_out[p]).wait()

            row = rows[p]
            G = 8
            for j0 in range(0, K // L, G):
                idxs = [ids_v[(j0 + t) // 8, pl.ds(((j0 + t) % 8) * L, L)]
                        for t in range(G)]
                vals = [plsc.load_gather(
                            row, [jnp.right_shift(ix, 7),
                                  jnp.bitwise_and(ix, 127)])
                        for ix in idxs]
                for t in range(G):
                    ob[pl.ds((j0 + t) * L, L)] = vals[t]
            pltpu.async_copy(ob, out_hbm.at[bo, c], sems_out[p])

        def chan_body(ci, _):
            c = c0 + ci * _NBUF
            for p in range(_NBUF):
                gather_row(p, c + p)
            return 0

        lax.fori_loop(0, CPW // _NBUF, chan_body, 0)
        for p in range(_NBUF):
            pltpu.make_async_copy(outs[p], out_hbm.at[bo, c0],
                                  sems_out[p]).wait()

    return gather_kernel(feats, ids3)


def _mlp_body(g_ref, w1_ref, b1_ref, w2_ref, b2_ref, out_ref):
    g = g_ref[0]
    h = lax.dot_general(g, w1_ref[...], (((0,), (0,)), ((), ())),
                        preferred_element_type=jnp.float32)
    h = jnp.maximum(h + b1_ref[...], 0.0)
    p = jnp.dot(h, w2_ref[...], preferred_element_type=jnp.float32)
    p = p + b2_ref[...]
    nrm = jnp.sqrt(jnp.sum(p * p, axis=1, keepdims=True))
    out_ref[...] = p / jnp.maximum(nrm, 1e-12)


def _mlp_alias_body(g_ref, w1_ref, b1_ref, w2_ref, b2_ref, prev_ref, out_ref):
    del prev_ref
    _mlp_body(g_ref, w1_ref, b1_ref, w2_ref, b2_ref, out_ref)


def _tc_mlp(gT, W1, b1, W2, b2, nrows, block_base, prev=None):
    bpc, C, K = gT.shape
    P = W1.shape[1]
    b1r = b1.reshape(1, P)
    b2r = b2.reshape(1, P)
    in_specs = [
        pl.BlockSpec((1, C, K), lambda b: (b, 0, 0)),
        pl.BlockSpec((C, P), lambda b: (0, 0)),
        pl.BlockSpec((1, P), lambda b: (0, 0)),
        pl.BlockSpec((P, P), lambda b: (0, 0)),
        pl.BlockSpec((1, P), lambda b: (0, 0)),
    ]
    args = [gT, W1, b1r, W2, b2r]
    kwargs = {}
    body = _mlp_body
    if prev is not None:
        in_specs.append(pl.BlockSpec(memory_space=pl.ANY))
        args.append(prev)
        kwargs["input_output_aliases"] = {5: 0}
        body = _mlp_alias_body

    return pl.pallas_call(
        body,
        grid=(bpc,),
        in_specs=in_specs,
        out_specs=pl.BlockSpec((K, P), lambda b: (block_base + b, 0)),
        out_shape=jax.ShapeDtypeStruct((nrows, P), jnp.float32),
        **kwargs,
    )(*args)


def kernel(feats, patch_ids, num_patches, W1, b1, W2, b2):
    B, C, H, W = feats.shape
    K = patch_ids.shape[1]
    ids3 = patch_ids.reshape(B, K // 128, 128)
    bpc = B // _NCHUNK
    gs = [_sc_gather(feats, ids3, i * bpc, bpc) for i in range(_NCHUNK)]
    p = _tc_mlp(gs[0], W1, b1, W2, b2, B * K, 0)
    for i in range(1, _NCHUNK):
        p = _tc_mlp(gs[i], W1, b1, W2, b2, B * K, i * bpc, prev=p)
    return (p, patch_ids)


# speedup vs baseline: 1.1047x; 1.0035x over previous
import functools

import jax
import jax.numpy as jnp
from jax import lax
from jax.experimental import pallas as pl
from jax.experimental.pallas import tpu as pltpu
from jax.experimental.pallas import tpu_sc as plsc

_NBUF = 6
_NCHUNK = 1


def _sc_gather(feats, ids3, b_base, bpc):
    B, C, H, W = feats.shape
    KH = ids3.shape[1]
    K = KH * 128
    info = plsc.get_sparse_core_info()
    NC, NS, L = info.num_cores, info.num_subcores, info.num_lanes
    NW = NC * NS
    assert NW % bpc == 0
    WPB = NW // bpc
    CPW = C // WPB
    assert CPW * WPB == C and CPW % _NBUF == 0
    mesh = plsc.VectorSubcoreMesh(core_axis_name="c", subcore_axis_name="s")

    @functools.partial(
        pl.kernel,
        mesh=mesh,
        out_type=jax.ShapeDtypeStruct((bpc, C, K), jnp.float32),
        scratch_types=[
            pltpu.VMEM((KH, 128), jnp.int32),
            [pltpu.VMEM((H, W), jnp.float32) for _ in range(_NBUF)],
            [pltpu.VMEM((K,), jnp.float32) for _ in range(_NBUF)],
            [pltpu.SemaphoreType.DMA for _ in range(_NBUF)],
            [pltpu.SemaphoreType.DMA for _ in range(_NBUF)],
        ],
        compiler_params=pltpu.CompilerParams(
            needs_layout_passes=False, use_tc_tiling_on_sc=True),
    )
    def gather_kernel(feats_hbm, ids_hbm, out_hbm, ids_v, rows, outs,
                      sems_in, sems_out):
        wid = lax.axis_index("s") * NC + lax.axis_index("c")
        bo = wid // WPB
        b = b_base + bo
        c0 = (wid % WPB) * CPW
        pltpu.sync_copy(ids_hbm.at[b], ids_v)
        for p in range(_NBUF - 1):
            pltpu.async_copy(feats_hbm.at[b, c0 + p], rows[p], sems_in[p])

        def gather_row(p, c):
            pltpu.make_async_copy(feats_hbm.at[b, c], rows[p],
                                  sems_in[p]).wait()

            @pl.when(c + _NBUF - 1 < c0 + CPW)
            def _():
                pltpu.async_copy(feats_hbm.at[b, c + _NBUF - 1],
                                 rows[(p + _NBUF - 1) % _NBUF],
                                 sems_in[(p + _NBUF - 1) % _NBUF])

            ob = outs[p]

            @pl.when(c - _NBUF >= c0)
            def _():
                pltpu.make_async_copy(ob, out_hbm.at[bo, c],
                                      sems_out[p]).wait()

            row = rows[p]
            G = 8
            for j0 in range(0, K // L, G):
                idxs = [ids_v[(j0 + t) // 8, pl.ds(((j0 + t) % 8) * L, L)]
                        for t in range(G)]
                vals = [plsc.load_gather(
                            row, [jnp.right_shift(ix, 7),
                                  jnp.bitwise_and(ix, 127)])
                        for ix in idxs]
                for t in range(G):
                    ob[pl.ds((j0 + t) * L, L)] = vals[t]
            pltpu.async_copy(ob, out_hbm.at[bo, c], sems_out[p])

        def chan_body(ci, _):
            c = c0 + ci * _NBUF
            for p in range(_NBUF):
                gather_row(p, c + p)
            return 0

        lax.fori_loop(0, CPW // _NBUF, chan_body, 0)
        for p in range(_NBUF):
            pltpu.make_async_copy(outs[p], out_hbm.at[bo, c0],
                                  sems_out[p]).wait()

    return gather_kernel(feats, ids3)


def _mlp_body(g_ref, w1_ref, b1_ref, w2_ref, b2_ref, out_ref):
    bf = jnp.bfloat16
    g = g_ref[0].astype(bf)
    h = lax.dot_general(g, w1_ref[...].astype(bf), (((0,), (0,)), ((), ())),
                        preferred_element_type=jnp.float32)
    h = jnp.maximum(h + b1_ref[...], 0.0)
    p = jnp.dot(h.astype(bf), w2_ref[...].astype(bf),
                preferred_element_type=jnp.float32)
    p = p + b2_ref[...]
    nrm = jnp.sqrt(jnp.sum(p * p, axis=1, keepdims=True))
    out_ref[...] = p / jnp.maximum(nrm, 1e-12)


def _mlp_alias_body(g_ref, w1_ref, b1_ref, w2_ref, b2_ref, prev_ref, out_ref):
    del prev_ref
    _mlp_body(g_ref, w1_ref, b1_ref, w2_ref, b2_ref, out_ref)


def _tc_mlp(gT, W1, b1, W2, b2, nrows, block_base, prev=None):
    bpc, C, K = gT.shape
    P = W1.shape[1]
    b1r = b1.reshape(1, P)
    b2r = b2.reshape(1, P)
    in_specs = [
        pl.BlockSpec((1, C, K), lambda b: (b, 0, 0)),
        pl.BlockSpec((C, P), lambda b: (0, 0)),
        pl.BlockSpec((1, P), lambda b: (0, 0)),
        pl.BlockSpec((P, P), lambda b: (0, 0)),
        pl.BlockSpec((1, P), lambda b: (0, 0)),
    ]
    args = [gT, W1, b1r, W2, b2r]
    kwargs = {}
    body = _mlp_body
    if prev is not None:
        in_specs.append(pl.BlockSpec(memory_space=pl.ANY))
        args.append(prev)
        kwargs["input_output_aliases"] = {5: 0}
        body = _mlp_alias_body

    return pl.pallas_call(
        body,
        grid=(bpc,),
        in_specs=in_specs,
        out_specs=pl.BlockSpec((K, P), lambda b: (block_base + b, 0)),
        out_shape=jax.ShapeDtypeStruct((nrows, P), jnp.float32),
        **kwargs,
    )(*args)


def kernel(feats, patch_ids, num_patches, W1, b1, W2, b2):
    B, C, H, W = feats.shape
    K = patch_ids.shape[1]
    ids3 = patch_ids.reshape(B, K // 128, 128)
    bpc = B // _NCHUNK
    gs = [_sc_gather(feats, ids3, i * bpc, bpc) for i in range(_NCHUNK)]
    p = _tc_mlp(gs[0], W1, b1, W2, b2, B * K, 0)
    for i in range(1, _NCHUNK):
        p = _tc_mlp(gs[i], W1, b1, W2, b2, B * K, i * bpc, prev=p)
    return (p, patch_ids)


# speedup vs baseline: 1.1119x; 1.0066x over previous
import functools

import jax
import jax.numpy as jnp
from jax import lax
from jax.experimental import pallas as pl
from jax.experimental.pallas import tpu as pltpu
from jax.experimental.pallas import tpu_sc as plsc

_NBUF = 6
_NCHUNK = 1


def _sc_gather(feats, ids2, b_base, bpc):
    B, C, H, W = feats.shape
    K = ids2.shape[1]
    info = plsc.get_sparse_core_info()
    NC, NS, L = info.num_cores, info.num_subcores, info.num_lanes
    NW = NC * NS
    assert NW % bpc == 0
    WPB = NW // bpc
    CPW = C // WPB
    assert CPW * WPB == C and CPW % _NBUF == 0
    mesh = plsc.VectorSubcoreMesh(core_axis_name="c", subcore_axis_name="s")

    @functools.partial(
        pl.kernel,
        mesh=mesh,
        out_type=jax.ShapeDtypeStruct((bpc, C, K), jnp.float32),
        scratch_types=[
            pltpu.VMEM((K,), jnp.int32),
            [pltpu.VMEM((H, W), jnp.float32) for _ in range(_NBUF)],
            [pltpu.VMEM((K,), jnp.float32) for _ in range(_NBUF)],
            [pltpu.SemaphoreType.DMA for _ in range(_NBUF)],
            [pltpu.SemaphoreType.DMA for _ in range(_NBUF)],
        ],
        compiler_params=pltpu.CompilerParams(
            needs_layout_passes=False, use_tc_tiling_on_sc=True),
    )
    def gather_kernel(feats_hbm, ids_hbm, out_hbm, ids_v, rows, outs,
                      sems_in, sems_out):
        wid = lax.axis_index("s") * NC + lax.axis_index("c")
        bo = wid // WPB
        b = b_base + bo
        c0 = (wid % WPB) * CPW
        pltpu.sync_copy(ids_hbm.at[b], ids_v)
        for p in range(_NBUF - 1):
            pltpu.async_copy(feats_hbm.at[b, c0 + p], rows[p], sems_in[p])

        def gather_row(p, c):
            pltpu.make_async_copy(feats_hbm.at[b, c], rows[p],
                                  sems_in[p]).wait()

            @pl.when(c + _NBUF - 1 < c0 + CPW)
            def _():
                pltpu.async_copy(feats_hbm.at[b, c + _NBUF - 1],
                                 rows[(p + _NBUF - 1) % _NBUF],
                                 sems_in[(p + _NBUF - 1) % _NBUF])

            ob = outs[p]

            @pl.when(c - _NBUF >= c0)
            def _():
                pltpu.make_async_copy(ob, out_hbm.at[bo, c],
                                      sems_out[p]).wait()

            row = rows[p]
            G = 8
            for j0 in range(0, K // L, G):
                idxs = [ids_v[pl.ds((j0 + t) * L, L)] for t in range(G)]
                vals = [plsc.load_gather(
                            row, [jnp.right_shift(ix, 7),
                                  jnp.bitwise_and(ix, 127)])
                        for ix in idxs]
                for t in range(G):
                    ob[pl.ds((j0 + t) * L, L)] = vals[t]
            pltpu.async_copy(ob, out_hbm.at[bo, c], sems_out[p])

        def chan_body(ci, _):
            c = c0 + ci * _NBUF
            for p in range(_NBUF):
                gather_row(p, c + p)
            return 0

        lax.fori_loop(0, CPW // _NBUF, chan_body, 0)
        for p in range(_NBUF):
            pltpu.make_async_copy(outs[p], out_hbm.at[bo, c0],
                                  sems_out[p]).wait()

    return gather_kernel(feats, ids2)


def _mlp_body(g_ref, w1_ref, b1_ref, w2_ref, b2_ref, out_ref):
    bf = jnp.bfloat16
    g = g_ref[0].astype(bf)
    h = lax.dot_general(g, w1_ref[...].astype(bf), (((0,), (0,)), ((), ())),
                        preferred_element_type=jnp.float32)
    h = jnp.maximum(h + b1_ref[...], 0.0)
    p = jnp.dot(h.astype(bf), w2_ref[...].astype(bf),
                preferred_element_type=jnp.float32)
    p = p + b2_ref[...]
    nrm = jnp.sqrt(jnp.sum(p * p, axis=1, keepdims=True))
    out_ref[...] = p / jnp.maximum(nrm, 1e-12)


def _mlp_alias_body(g_ref, w1_ref, b1_ref, w2_ref, b2_ref, prev_ref, out_ref):
    del prev_ref
    _mlp_body(g_ref, w1_ref, b1_ref, w2_ref, b2_ref, out_ref)


def _tc_mlp(gT, W1, b1, W2, b2, nrows, block_base, prev=None):
    bpc, C, K = gT.shape
    P = W1.shape[1]
    b1r = b1.reshape(1, P)
    b2r = b2.reshape(1, P)
    in_specs = [
        pl.BlockSpec((1, C, K), lambda b: (b, 0, 0)),
        pl.BlockSpec((C, P), lambda b: (0, 0)),
        pl.BlockSpec((1, P), lambda b: (0, 0)),
        pl.BlockSpec((P, P), lambda b: (0, 0)),
        pl.BlockSpec((1, P), lambda b: (0, 0)),
    ]
    args = [gT, W1, b1r, W2, b2r]
    kwargs = {}
    body = _mlp_body
    if prev is not None:
        in_specs.append(pl.BlockSpec(memory_space=pl.ANY))
        args.append(prev)
        kwargs["input_output_aliases"] = {5: 0}
        body = _mlp_alias_body

    return pl.pallas_call(
        body,
        grid=(bpc,),
        in_specs=in_specs,
        out_specs=pl.BlockSpec((K, P), lambda b: (block_base + b, 0)),
        out_shape=jax.ShapeDtypeStruct((nrows, P), jnp.float32),
        **kwargs,
    )(*args)


def kernel(feats, patch_ids, num_patches, W1, b1, W2, b2):
    B, C, H, W = feats.shape
    K = patch_ids.shape[1]
    bpc = B // _NCHUNK
    gs = [_sc_gather(feats, patch_ids, i * bpc, bpc) for i in range(_NCHUNK)]
    p = _tc_mlp(gs[0], W1, b1, W2, b2, B * K, 0)
    for i in range(1, _NCHUNK):
        p = _tc_mlp(gs[i], W1, b1, W2, b2, B * K, i * bpc, prev=p)
    return (p, patch_ids)
